# baseline (speedup 1.0000x reference)
import functools

import jax
import jax.numpy as jnp
from jax import lax
from jax.experimental import pallas as pl
from jax.experimental.pallas import tpu as pltpu
from jax.experimental.pallas import tpu_sc as plsc

EXER_N = 10000
N_EX = 16384
KN = 64
NC = 2
NS = 16
LANES = 16

EB = 1024
EGRID = (EXER_N + EB - 1) // EB
OUT_PER_TILE = N_EX // (NC * NS)
IDX_MINOR = 128
IDX_ROWS = OUT_PER_TILE // IDX_MINOR


def _dense_body(stu_ref, ut_ref, qt_ref, dt_ref, at_ref, gt_ref,
                pred_ref, state_ref):
    col = stu_ref[0] % 128
    lane = lax.broadcasted_iota(jnp.int32, (KN, 128), 1)
    u = jnp.sum(jnp.where(lane == col, ut_ref[...], 0.0), axis=1,
                keepdims=True)
    state_ref[...] = u
    v = jnp.sum(qt_ref[...] * u, axis=0, keepdims=True)
    x = at_ref[...] * (v / dt_ref[...] - 0.5) + gt_ref[...]
    pred_ref[...] = (1.0 / (1.0 + jnp.exp(x))).reshape(EB)


@jax.jit
def _dense_call(stu_id, U_T, q_T, d_T, a_T, g_T):
    grid_spec = pltpu.PrefetchScalarGridSpec(
        num_scalar_prefetch=1,
        grid=(EGRID,),
        in_specs=[
            pl.BlockSpec((KN, 128), lambda i, stu: (0, stu[0] // 128)),
            pl.BlockSpec((KN, EB), lambda i, stu: (0, i)),
            pl.BlockSpec((1, EB), lambda i, stu: (0, i)),
            pl.BlockSpec((1, EB), lambda i, stu: (0, i)),
            pl.BlockSpec((1, EB), lambda i, stu: (0, i)),
        ],
        out_specs=[
            pl.BlockSpec((EB,), lambda i, stu: (i,)),
            pl.BlockSpec((KN, 1), lambda i, stu: (0, 0)),
        ],
    )
    return pl.pallas_call(
        _dense_body,
        grid_spec=grid_spec,
        out_shape=(
            jax.ShapeDtypeStruct((EXER_N,), jnp.float32),
            jax.ShapeDtypeStruct((KN, 1), jnp.float32),
        ),
    )(stu_id, U_T, q_T, d_T, a_T, g_T)


def _sc_body(ex_hbm, tab_hbm, pred_out, exidx_v, psel_v, sem):
    c = lax.axis_index("c")
    s = lax.axis_index("s")
    wid = s * NC + c
    pltpu.sync_copy(ex_hbm.at[pl.ds(wid * OUT_PER_TILE, OUT_PER_TILE)],
                    exidx_v)
    handles = [
        pltpu.async_copy(
            tab_hbm.at[exidx_v.at[pl.ds(j * IDX_MINOR, IDX_MINOR)]],
            psel_v.at[pl.ds(j * IDX_MINOR, IDX_MINOR)], sem)
        for j in range(IDX_ROWS)
    ]
    for h in handles:
        h.wait()
    pltpu.sync_copy(psel_v, pred_out.at[pl.ds(wid * OUT_PER_TILE,
                                              OUT_PER_TILE)])


@jax.jit
def _sc_call(ex_id, table):
    mesh = plsc.VectorSubcoreMesh(core_axis_name="c", subcore_axis_name="s")
    run = functools.partial(
        pl.kernel,
        mesh=mesh,
        compiler_params=pltpu.CompilerParams(
            needs_layout_passes=False, use_tc_tiling_on_sc=False,
            skip_device_barrier=True),
        out_type=jax.ShapeDtypeStruct((N_EX,), jnp.float32),
        scratch_types=[
            pltpu.VMEM((OUT_PER_TILE,), jnp.int32),
            pltpu.VMEM((OUT_PER_TILE,), jnp.float32),
            pltpu.SemaphoreType.DMA,
        ],
    )(_sc_body)
    return run(ex_id, table)


def kernel(stu_id, kn_id, score, user_k_kc, ex_id, q_kn, d, U_embeding,
           alpha, gamma_e):
    table, state = _dense_call(
        stu_id.astype(jnp.int32), U_embeding.T, q_kn.T, d.T, alpha.T,
        gamma_e.T)
    pred_flat = _sc_call(ex_id.astype(jnp.int32), table)
    return state, pred_flat.reshape(N_EX, 1)


# speedup vs baseline: 1.1835x; 1.1835x over previous
import functools

import jax
import jax.numpy as jnp
from jax import lax
from jax.experimental import pallas as pl
from jax.experimental.pallas import tpu as pltpu
from jax.experimental.pallas import tpu_sc as plsc

EXER_N = 10000
N_EX = 16384
KN = 64
NC = 2
NS = 16
LANES = 16

EB = 5120
EGRID = (EXER_N + EB - 1) // EB
OUT_PER_TILE = N_EX // (NC * NS)
IDX_MINOR = 128
IDX_ROWS = OUT_PER_TILE // IDX_MINOR


def _dense_body(stu_ref, ut_ref, qt_ref, dt_ref, at_ref, gt_ref,
                pred_ref, state_ref):
    col = stu_ref[0] % 128
    lane = lax.broadcasted_iota(jnp.int32, (KN, 128), 1)
    u = jnp.sum(jnp.where(lane == col, ut_ref[...], 0.0), axis=1,
                keepdims=True)
    state_ref[...] = u
    v = jnp.sum(qt_ref[...] * u, axis=0, keepdims=True)
    x = at_ref[...] * (v / dt_ref[...] - 0.5) + gt_ref[...]
    pred_ref[...] = (1.0 / (1.0 + jnp.exp(x))).reshape(EB)


@jax.jit
def _dense_call(stu_id, U_T, q_T, d_T, a_T, g_T):
    grid_spec = pltpu.PrefetchScalarGridSpec(
        num_scalar_prefetch=1,
        grid=(EGRID,),
        in_specs=[
            pl.BlockSpec((KN, 128), lambda i, stu: (0, stu[0] // 128)),
            pl.BlockSpec((KN, EB), lambda i, stu: (0, i)),
            pl.BlockSpec((1, EB), lambda i, stu: (0, i)),
            pl.BlockSpec((1, EB), lambda i, stu: (0, i)),
            pl.BlockSpec((1, EB), lambda i, stu: (0, i)),
        ],
        out_specs=[
            pl.BlockSpec((EB,), lambda i, stu: (i,)),
            pl.BlockSpec((KN, 1), lambda i, stu: (0, 0)),
        ],
    )
    return pl.pallas_call(
        _dense_body,
        grid_spec=grid_spec,
        out_shape=(
            jax.ShapeDtypeStruct((EXER_N,), jnp.float32),
            jax.ShapeDtypeStruct((KN, 1), jnp.float32),
        ),
    )(stu_id, U_T, q_T, d_T, a_T, g_T)


def _sc_body(ex_hbm, tab_hbm, pred_out, exidx_v, psel_v, sem):
    c = lax.axis_index("c")
    s = lax.axis_index("s")
    wid = s * NC + c
    pltpu.sync_copy(ex_hbm.at[pl.ds(wid * OUT_PER_TILE, OUT_PER_TILE)],
                    exidx_v)
    handles = [
        pltpu.async_copy(
            tab_hbm.at[exidx_v.at[pl.ds(j * IDX_MINOR, IDX_MINOR)]],
            psel_v.at[pl.ds(j * IDX_MINOR, IDX_MINOR)], sem)
        for j in range(IDX_ROWS)
    ]
    for h in handles:
        h.wait()
    pltpu.sync_copy(psel_v, pred_out.at[pl.ds(wid * OUT_PER_TILE,
                                              OUT_PER_TILE)])


@jax.jit
def _sc_call(ex_id, table):
    mesh = plsc.VectorSubcoreMesh(core_axis_name="c", subcore_axis_name="s")
    run = functools.partial(
        pl.kernel,
        mesh=mesh,
        compiler_params=pltpu.CompilerParams(
            needs_layout_passes=False, use_tc_tiling_on_sc=False,
            skip_device_barrier=True),
        out_type=jax.ShapeDtypeStruct((N_EX,), jnp.float32),
        scratch_types=[
            pltpu.VMEM((OUT_PER_TILE,), jnp.int32),
            pltpu.VMEM((OUT_PER_TILE,), jnp.float32),
            pltpu.SemaphoreType.DMA,
        ],
    )(_sc_body)
    return run(ex_id, table)


def kernel(stu_id, kn_id, score, user_k_kc, ex_id, q_kn, d, U_embeding,
           alpha, gamma_e):
    table, state = _dense_call(
        stu_id.astype(jnp.int32), U_embeding.T, q_kn.T, d.T, alpha.T,
        gamma_e.T)
    pred_flat = _sc_call(ex_id.astype(jnp.int32), table)
    return state, pred_flat.reshape(N_EX, 1)
